# baseline (speedup 1.0000x reference)
import functools

import jax
import jax.numpy as jnp
from jax import lax
from jax.experimental import pallas as pl
from jax.experimental.pallas import tpu as pltpu
from jax.experimental.pallas import tpu_sc as plsc

NBUF = 4


@functools.lru_cache(maxsize=None)
def _make_lookup(S, T, D):
    info = plsc.get_sparse_core_info()
    NC, NS = info.num_cores, info.num_subcores
    NW = NC * NS
    assert S % NW == 0
    s_per_w = S // NW

    mesh = plsc.VectorSubcoreMesh(core_axis_name="c", subcore_axis_name="s")

    TP = (T + 15) // 16 * 16

    @functools.partial(
        pl.kernel,
        mesh=mesh,
        out_type=jax.ShapeDtypeStruct((S, T, D), jnp.float32),
        scratch_types=[
            pltpu.VMEM((s_per_w, TP), jnp.int32),
        ]
        + [pltpu.VMEM((TP, D), jnp.float32) for _ in range(NBUF)]
        + [pltpu.SemaphoreType.DMA for _ in range(2 * NBUF)],
        compiler_params=pltpu.CompilerParams(use_tc_tiling_on_sc=False),
    )
    def lookup(idx_hbm, table_hbm, out_hbm, idx_v, *bufs_sems):
        bufs = bufs_sems[:NBUF]
        gsems = bufs_sems[NBUF : 2 * NBUF]
        osems = bufs_sems[2 * NBUF :]

        wid = lax.axis_index("s") * NC + lax.axis_index("c")
        base = wid * s_per_w
        pltpu.sync_copy(idx_hbm.at[wid], idx_v)

        def start_gather(g, b):
            pltpu.async_copy(table_hbm.at[idx_v.at[g]], bufs[b], gsems[b])

        def wait_gather(g, b):
            pltpu.make_async_copy(
                table_hbm.at[idx_v.at[g]], bufs[b], gsems[b]
            ).wait()

        def start_write(g, b):
            pltpu.async_copy(
                bufs[b].at[pl.ds(0, T)], out_hbm.at[base + g], osems[b]
            )

        def wait_write(g, b):
            pltpu.make_async_copy(
                bufs[b].at[pl.ds(0, T)], out_hbm.at[base + g], osems[b]
            ).wait()

        for k in range(NBUF - 1):
            start_gather(k, k)

        wait_gather(0, 0)
        start_write(0, 0)
        start_gather(NBUF - 1, NBUF - 1)

        n_main = s_per_w - NBUF
        assert n_main % NBUF == 0

        def quad(q, carry):
            for j in range(NBUF):
                g = 1 + q * NBUF + j
                b = (1 + j) % NBUF
                wait_gather(g, b)
                start_write(g, b)
                wait_write(g - 1, (b - 1) % NBUF)
                start_gather(g + NBUF - 1, (b - 1) % NBUF)
            return carry

        lax.fori_loop(0, n_main // NBUF, quad, 0)

        for g in range(s_per_w - NBUF + 1, s_per_w):
            b = g % NBUF
            wait_gather(g, b)
            start_write(g, b)
            wait_write(g - 1, (b - 1) % NBUF)
        wait_write(s_per_w - 1, (s_per_w - 1) % NBUF)

    return lookup


def kernel(input_ids, embed_weight):
    S, T = input_ids.shape
    D = embed_weight.shape[1]
    info = plsc.get_sparse_core_info()
    NW = info.num_cores * info.num_subcores
    idx = input_ids.reshape(NW, S // NW, T).astype(jnp.int32)
    TP = (T + 15) // 16 * 16
    if TP != T:
        idx = jnp.pad(idx, ((0, 0), (0, 0), (0, TP - T)))
    return _make_lookup(S, T, D)(idx, embed_weight)


# speedup vs baseline: 3.2949x; 3.2949x over previous
import functools

import jax
import jax.numpy as jnp
from jax import lax
from jax.experimental import pallas as pl
from jax.experimental.pallas import tpu as pltpu
from jax.experimental.pallas import tpu_sc as plsc

NBUF = 4
CHUNK = 64


@functools.lru_cache(maxsize=None)
def _make_lookup(B, D):
    info = plsc.get_sparse_core_info()
    NC, NS = info.num_cores, info.num_subcores
    NW = NC * NS
    assert B % (NW * CHUNK) == 0
    b_per_w = B // NW
    n = b_per_w // CHUNK
    assert (n - NBUF) % NBUF == 0

    mesh = plsc.VectorSubcoreMesh(core_axis_name="c", subcore_axis_name="s")

    @functools.partial(
        pl.kernel,
        mesh=mesh,
        out_type=jax.ShapeDtypeStruct((B, D), jnp.float32),
        scratch_types=[
            pltpu.VMEM((n, CHUNK), jnp.int32),
        ]
        + [pltpu.VMEM((CHUNK, D), jnp.float32) for _ in range(NBUF)]
        + [pltpu.SemaphoreType.DMA for _ in range(2 * NBUF)],
    )
    def lookup(idx_hbm, table_hbm, out_hbm, idx_v, *bufs_sems):
        bufs = bufs_sems[:NBUF]
        gsems = bufs_sems[NBUF : 2 * NBUF]
        osems = bufs_sems[2 * NBUF :]

        wid = lax.axis_index("s") * NC + lax.axis_index("c")
        base = wid * b_per_w
        pltpu.sync_copy(idx_hbm.at[wid], idx_v)

        def start_gather(g, b):
            pltpu.async_copy(
                table_hbm.at[idx_v.at[g]], bufs[b], gsems[b]
            )

        def wait_gather(g, b):
            pltpu.make_async_copy(
                table_hbm.at[idx_v.at[g]], bufs[b], gsems[b]
            ).wait()

        def start_write(g, b):
            pltpu.async_copy(
                bufs[b], out_hbm.at[pl.ds(base + g * CHUNK, CHUNK)], osems[b]
            )

        def wait_write(g, b):
            pltpu.make_async_copy(
                bufs[b], out_hbm.at[pl.ds(base + g * CHUNK, CHUNK)], osems[b]
            ).wait()

        for k in range(NBUF - 1):
            start_gather(k, k)

        wait_gather(0, 0)
        start_write(0, 0)
        start_gather(NBUF - 1, NBUF - 1)

        def quad(q, carry):
            for j in range(NBUF):
                g = 1 + q * NBUF + j
                b = (1 + j) % NBUF
                wait_gather(g, b)
                start_write(g, b)
                wait_write(g - 1, (b - 1) % NBUF)
                start_gather(g + NBUF - 1, (b - 1) % NBUF)
            return carry

        lax.fori_loop(0, (n - NBUF) // NBUF, quad, 0)

        for g in range(n - NBUF + 1, n):
            b = g % NBUF
            wait_gather(g, b)
            start_write(g, b)
            wait_write(g - 1, (b - 1) % NBUF)
        wait_write(n - 1, (n - 1) % NBUF)

    return lookup


def kernel(input_ids, embed_weight):
    B = input_ids.shape[0] * input_ids.shape[1]
    D = embed_weight.shape[1]
    info = plsc.get_sparse_core_info()
    NW = info.num_cores * info.num_subcores
    idx = input_ids.reshape(NW, (B // NW) // CHUNK, CHUNK).astype(jnp.int32)
    out = _make_lookup(B, D)(idx, embed_weight)
    return out.reshape(input_ids.shape[0], input_ids.shape[1], D)
